# baseline (speedup 1.0000x reference)
import functools

import jax
import jax.numpy as jnp
import numpy as np
from jax import lax
from jax.experimental import pallas as pl
from jax.experimental.pallas import tpu as pltpu
from jax.experimental.pallas import tpu_sc as plsc

N = 10000
F_IN = 128
H1 = 8
D1 = 8
C1 = H1 * D1
NPAD = 10240
BN = 1280
ROW1 = 80
ROW2 = 16
RDST = 16

NC = 2
NS = 16
TILES = NC * NS
CH = 128
EP_RAW = 320000 + N
SB = 2
K1 = 84
NSUP = K1 // SB
EPAD = TILES * CH * K1
RPT = NPAD // NS


def _prep_kernel(x_ref, w1_ref, as1_ref, ad1_ref, t1_ref, td_ref):
    h = jnp.dot(x_ref[...], w1_ref[...], preferred_element_type=jnp.float32)
    s = jnp.dot(h, as1_ref[...], preferred_element_type=jnp.float32)
    d = jnp.dot(h, ad1_ref[...], preferred_element_type=jnp.float32)
    z8 = jnp.zeros((h.shape[0], 8), jnp.float32)
    t1_ref[...] = jnp.concatenate([h, s, z8], axis=1)
    td_ref[...] = jnp.concatenate([d, z8], axis=1)


def _mid_kernel(p0_ref, p1_ref, b1_ref, w2_ref, ws2_ref, wd2_ref, r_ref, t2_ref):
    a = p0_ref[...] + p1_ref[...]
    num = a[:, 0:C1]
    den = a[:, C1:C1 + H1]
    denr = jnp.dot(den, r_ref[...], preferred_element_type=jnp.float32)
    out1 = num / (denr + 1e-16) + b1_ref[...]
    g = jnp.where(out1 > 0, out1, jnp.exp(jnp.minimum(out1, 0.0)) - 1.0)
    h2 = jnp.dot(g, w2_ref[...], preferred_element_type=jnp.float32)
    s2 = jnp.dot(g, ws2_ref[...], preferred_element_type=jnp.float32)
    d2 = jnp.dot(g, wd2_ref[...], preferred_element_type=jnp.float32)
    z12 = jnp.zeros((a.shape[0], 12), jnp.float32)
    t2_ref[...] = jnp.concatenate([h2, s2, d2, z12], axis=1)


def _final_kernel(q0_ref, q1_ref, b2_ref, o_ref):
    a = q0_ref[...] + q1_ref[...]
    num = a[:, 0:2]
    den = a[:, 2:3]
    o = num / (den + 1e-16) + b2_ref[...]
    m = jnp.max(o, axis=1, keepdims=True)
    lse = m + jnp.log(jnp.sum(jnp.exp(o - m), axis=1, keepdims=True))
    o_ref[...] = o - lse


def _leaky_exp(x):
    return jnp.exp(jnp.where(x >= 0, x, x * 0.2))


_U = 4


def _permute(vec, idx):
    return vec.at[idx].get(mode="promise_in_bounds")


def _group1(srows, drows, base):
    lanes = lax.iota(jnp.int32, 16)
    for u in range(_U):
        e = base + u
        al = srows[e, pl.ds(C1, 16)] + drows[e]
        p16 = jnp.exp(jnp.where(al >= 0.0, al, al * 0.2))
        srows[e, pl.ds(C1, 16)] = p16
        for v in range(4):
            prep = _permute(p16, lanes // 8 + 2 * v)
            srows[e, pl.ds(16 * v, 16)] = srows[e, pl.ds(16 * v, 16)] * prep


def _group2(srows, drows, base):
    lanes = lax.iota(jnp.int32, 16)
    for u in range(_U):
        e = base + u
        sr = srows[e]
        dr = drows[e]
        al = _permute(sr, lanes * 0 + 2) + _permute(dr, lanes * 0 + 3)
        p = jnp.exp(jnp.where(al >= 0.0, al, al * 0.2))
        srows[e] = jnp.where(lanes == 2, p, sr * p)


def _make_edge_body(group_fn):
    def body(tsrc_hbm, tdst_hbm, src_hbm, dst_hbm, zero_hbm, out_hbm,
             sidx, didx, sA, dA, sB, dB, acc, gsemA, gsemB):
        c = lax.axis_index("c")
        s = lax.axis_index("s")
        wid = s * NC + c
        pltpu.sync_copy(zero_hbm.at[pl.ds(s * RPT, RPT)], acc.at[pl.ds(s * RPT, RPT)])
        plsc.subcore_barrier()
        pltpu.sync_copy(src_hbm.at[wid], sidx)
        pltpu.sync_copy(dst_hbm.at[wid], didx)

        def fire(kk, srows, drows, gsem):
            for j in range(SB):
                pltpu.async_copy(tsrc_hbm.at[sidx.at[kk + j]],
                                 srows.at[pl.ds(j * CH, CH)], gsem)
                pltpu.async_copy(tdst_hbm.at[didx.at[kk + j]],
                                 drows.at[pl.ds(j * CH, CH)], gsem)

        def drain(kk, srows, drows, gsem):
            for j in range(SB):
                pltpu.make_async_copy(tsrc_hbm.at[sidx.at[kk + j]],
                                      srows.at[pl.ds(j * CH, CH)], gsem).wait()
                pltpu.make_async_copy(tdst_hbm.at[didx.at[kk + j]],
                                      drows.at[pl.ds(j * CH, CH)], gsem).wait()

        def process(kk, srows, drows):
            lax.fori_loop(
                0, SB * CH // _U,
                lambda i, cy: (group_fn(srows, drows, i * _U), cy)[1], 0)
            for j in range(SB):
                pltpu.sync_copy(srows.at[pl.ds(j * CH, CH)],
                                acc.at[didx.at[kk + j]], add=True)

        fire(0, sA, dA, gsemA)

        def pair(t, cy):
            kA = 2 * t * SB
            kB = kA + SB
            fire(kB, sB, dB, gsemB)
            drain(kA, sA, dA, gsemA)
            process(kA, sA, dA)

            @pl.when(t < NSUP // 2 - 1)
            def _():
                fire(kA + 2 * SB, sA, dA, gsemA)

            drain(kB, sB, dB, gsemB)
            process(kB, sB, dB)
            return cy

        lax.fori_loop(0, NSUP // 2, pair, 0)
        plsc.subcore_barrier()
        pltpu.sync_copy(acc.at[pl.ds(s * RPT, RPT)], out_hbm.at[c, pl.ds(s * RPT, RPT)])

    return body


_edge1_body = _make_edge_body(_group1)
_edge2_body = _make_edge_body(_group2)


_SC_MESH = plsc.VectorSubcoreMesh(core_axis_name="c", subcore_axis_name="s")
_SC_PARAMS = pltpu.CompilerParams(
    needs_layout_passes=False, use_tc_tiling_on_sc=False)

_edge1 = functools.partial(
    pl.kernel,
    out_type=jax.ShapeDtypeStruct((NC, NPAD, ROW1), jnp.float32),
    mesh=_SC_MESH,
    compiler_params=_SC_PARAMS,
    scratch_types=[
        pltpu.VMEM((K1, CH), jnp.int32),
        pltpu.VMEM((K1, CH), jnp.int32),
        pltpu.VMEM((SB * CH, ROW1), jnp.float32),
        pltpu.VMEM((SB * CH, RDST), jnp.float32),
        pltpu.VMEM((SB * CH, ROW1), jnp.float32),
        pltpu.VMEM((SB * CH, RDST), jnp.float32),
        pltpu.VMEM_SHARED((NPAD, ROW1), jnp.float32),
        pltpu.SemaphoreType.DMA,
        pltpu.SemaphoreType.DMA,
    ],
)(_edge1_body)

_edge2 = functools.partial(
    pl.kernel,
    out_type=jax.ShapeDtypeStruct((NC, NPAD, ROW2), jnp.float32),
    mesh=_SC_MESH,
    compiler_params=_SC_PARAMS,
    scratch_types=[
        pltpu.VMEM((K1, CH), jnp.int32),
        pltpu.VMEM((K1, CH), jnp.int32),
        pltpu.VMEM((SB * CH, ROW2), jnp.float32),
        pltpu.VMEM((SB * CH, ROW2), jnp.float32),
        pltpu.VMEM((SB * CH, ROW2), jnp.float32),
        pltpu.VMEM((SB * CH, ROW2), jnp.float32),
        pltpu.VMEM_SHARED((NPAD, ROW2), jnp.float32),
        pltpu.SemaphoreType.DMA,
        pltpu.SemaphoreType.DMA,
    ],
)(_edge2_body)


def kernel(x, edge_index, W1, att_src1, att_dst1, bias1, W2, att_src2, att_dst2, bias2):
    f32 = jnp.float32
    eye8 = jnp.eye(H1, dtype=f32)
    As1 = (att_src1.reshape(H1, D1)[:, :, None] * eye8[:, None, :]).reshape(C1, H1)
    Ad1 = (att_dst1.reshape(H1, D1)[:, :, None] * eye8[:, None, :]).reshape(C1, H1)
    R = jnp.repeat(eye8, D1, axis=1)
    Ws2 = W2 @ att_src2.reshape(2, 1)
    Wd2 = W2 @ att_dst2.reshape(2, 1)
    xp = jnp.pad(x, ((0, NPAD - N), (0, 0)))

    loop = jnp.arange(N, dtype=jnp.int32)
    padv = jnp.full((EPAD - EP_RAW,), N, jnp.int32)
    src = jnp.concatenate([edge_index[0], loop, padv]).reshape(TILES, K1, CH)
    dst = jnp.concatenate([edge_index[1], loop, padv]).reshape(TILES, K1, CH)

    zeros80 = jnp.zeros((NPAD, ROW1), f32)
    zeros16 = jnp.zeros((NPAD, ROW2), f32)

    grid = NPAD // BN
    t1, td = pl.pallas_call(
        _prep_kernel,
        grid=(grid,),
        in_specs=[
            pl.BlockSpec((BN, F_IN), lambda i: (i, 0)),
            pl.BlockSpec((F_IN, C1), lambda i: (0, 0)),
            pl.BlockSpec((C1, H1), lambda i: (0, 0)),
            pl.BlockSpec((C1, H1), lambda i: (0, 0)),
        ],
        out_specs=[
            pl.BlockSpec((BN, ROW1), lambda i: (i, 0)),
            pl.BlockSpec((BN, RDST), lambda i: (i, 0)),
        ],
        out_shape=[
            jax.ShapeDtypeStruct((NPAD, ROW1), f32),
            jax.ShapeDtypeStruct((NPAD, RDST), f32),
        ],
    )(xp, W1, As1, Ad1)

    parts1 = _edge1(t1, td, src, dst, zeros80)

    t2 = pl.pallas_call(
        _mid_kernel,
        grid=(grid,),
        in_specs=[
            pl.BlockSpec((BN, ROW1), lambda i: (i, 0)),
            pl.BlockSpec((BN, ROW1), lambda i: (i, 0)),
            pl.BlockSpec((1, C1), lambda i: (0, 0)),
            pl.BlockSpec((C1, 2), lambda i: (0, 0)),
            pl.BlockSpec((C1, 1), lambda i: (0, 0)),
            pl.BlockSpec((C1, 1), lambda i: (0, 0)),
            pl.BlockSpec((H1, C1), lambda i: (0, 0)),
        ],
        out_specs=pl.BlockSpec((BN, ROW2), lambda i: (i, 0)),
        out_shape=jax.ShapeDtypeStruct((NPAD, ROW2), f32),
    )(parts1[0], parts1[1], bias1.reshape(1, C1), W2, Ws2, Wd2, R)

    parts2 = _edge2(t2, t2, src, dst, zeros16)

    out = pl.pallas_call(
        _final_kernel,
        grid=(grid,),
        in_specs=[
            pl.BlockSpec((BN, ROW2), lambda i: (i, 0)),
            pl.BlockSpec((BN, ROW2), lambda i: (i, 0)),
            pl.BlockSpec((1, 2), lambda i: (0, 0)),
        ],
        out_specs=pl.BlockSpec((BN, 2), lambda i: (i, 0)),
        out_shape=jax.ShapeDtypeStruct((NPAD, 2), f32),
    )(parts2[0], parts2[1], bias2.reshape(1, 2))

    return out[:N]


# speedup vs baseline: 1.8106x; 1.8106x over previous
import functools

import jax
import jax.numpy as jnp
import numpy as np
from jax import lax
from jax.experimental import pallas as pl
from jax.experimental.pallas import tpu as pltpu
from jax.experimental.pallas import tpu_sc as plsc

N = 10000
F_IN = 128
H1 = 8
D1 = 8
C1 = H1 * D1
NPAD = 10240
BN = 1280
ROW1 = 80
ROW2 = 16
RDST = 16

NC = 2
NS = 16
TILES = NC * NS
CH = 128
EP_RAW = 320000 + N
SB = 2
K1 = 84
NSUP = K1 // SB
EPAD = TILES * CH * K1
RPT = NPAD // NS


def _prep_kernel(x_ref, w1_ref, as1_ref, ad1_ref, t1_ref, td_ref):
    h = jnp.dot(x_ref[...], w1_ref[...], preferred_element_type=jnp.float32)
    s = jnp.dot(h, as1_ref[...], preferred_element_type=jnp.float32)
    d = jnp.dot(h, ad1_ref[...], preferred_element_type=jnp.float32)
    z8 = jnp.zeros((h.shape[0], 8), jnp.float32)
    t1_ref[...] = jnp.concatenate([h, s, z8], axis=1)
    td_ref[...] = jnp.concatenate([d, z8], axis=1)


def _mid_kernel(p0_ref, p1_ref, b1_ref, w2_ref, ws2_ref, wd2_ref, r_ref, t2_ref):
    a = p0_ref[...] + p1_ref[...]
    num = a[:, 0:C1]
    den = a[:, C1:C1 + H1]
    denr = jnp.dot(den, r_ref[...], preferred_element_type=jnp.float32)
    out1 = num / (denr + 1e-16) + b1_ref[...]
    g = jnp.where(out1 > 0, out1, jnp.exp(jnp.minimum(out1, 0.0)) - 1.0)
    h2 = jnp.dot(g, w2_ref[...], preferred_element_type=jnp.float32)
    s2 = jnp.dot(g, ws2_ref[...], preferred_element_type=jnp.float32)
    d2 = jnp.dot(g, wd2_ref[...], preferred_element_type=jnp.float32)
    z12 = jnp.zeros((a.shape[0], 12), jnp.float32)
    t2_ref[...] = jnp.concatenate([h2, s2, d2, z12], axis=1)


def _final_kernel(q0_ref, q1_ref, b2_ref, o_ref):
    a = q0_ref[...] + q1_ref[...]
    num = a[:, 0:2]
    den = a[:, 2:3]
    o = num / (den + 1e-16) + b2_ref[...]
    m = jnp.max(o, axis=1, keepdims=True)
    lse = m + jnp.log(jnp.sum(jnp.exp(o - m), axis=1, keepdims=True))
    o_ref[...] = o - lse


def _leaky_exp(x):
    return jnp.exp(jnp.where(x >= 0, x, x * 0.2))


_U = 4


def _permute(vec, idx):
    return vec.at[idx].get(mode="promise_in_bounds")


def _group1(srows, drows, base):
    lanes = lax.iota(jnp.int32, 16)
    for u in range(_U):
        e = base + u
        al = srows[e, pl.ds(C1, 16)] + drows[e]
        p16 = jnp.exp(jnp.where(al >= 0.0, al, al * 0.2))
        srows[e, pl.ds(C1, 16)] = p16
        for v in range(4):
            prep = _permute(p16, lanes // 8 + 2 * v)
            srows[e, pl.ds(16 * v, 16)] = srows[e, pl.ds(16 * v, 16)] * prep


def _group2(srows, drows, base):
    lanes = lax.iota(jnp.int32, 16)
    for u in range(_U):
        e = base + u
        sr = srows[e]
        dr = drows[e]
        al = _permute(sr, lanes * 0 + 2) + _permute(dr, lanes * 0 + 3)
        p = jnp.exp(jnp.where(al >= 0.0, al, al * 0.2))
        srows[e] = jnp.where(lanes == 2, p, sr * p)


def _make_edge_body(group_fn):
    def body(tsrc_hbm, tdst_hbm, src_hbm, dst_hbm, zero_hbm, out_hbm,
             sidx, didx, sA, dA, sB, dB, acc, gsemA, gsemB):
        c = lax.axis_index("c")
        s = lax.axis_index("s")
        wid = s * NC + c
        pltpu.sync_copy(zero_hbm.at[pl.ds(s * RPT, RPT)], acc.at[pl.ds(s * RPT, RPT)])
        plsc.subcore_barrier()
        pltpu.sync_copy(src_hbm.at[wid], sidx)
        pltpu.sync_copy(dst_hbm.at[wid], didx)

        def fire(kk, srows, drows, gsem):
            for j in range(SB):
                pltpu.async_copy(tsrc_hbm.at[sidx.at[kk + j]],
                                 srows.at[pl.ds(j * CH, CH)], gsem)
                pltpu.async_copy(tdst_hbm.at[didx.at[kk + j]],
                                 drows.at[pl.ds(j * CH, CH)], gsem)

        def drain(kk, srows, drows, gsem):
            for j in range(SB):
                pltpu.make_async_copy(tsrc_hbm.at[sidx.at[kk + j]],
                                      srows.at[pl.ds(j * CH, CH)], gsem).wait()
                pltpu.make_async_copy(tdst_hbm.at[didx.at[kk + j]],
                                      drows.at[pl.ds(j * CH, CH)], gsem).wait()

        def process(kk, srows, drows):
            if group_fn is not None:
                lax.fori_loop(
                    0, SB * CH // _U,
                    lambda i, cy: (group_fn(srows, drows, i * _U), cy)[1], 0)
            for j in range(SB):
                pltpu.sync_copy(srows.at[pl.ds(j * CH, CH)],
                                acc.at[didx.at[kk + j]], add=True)

        fire(0, sA, dA, gsemA)

        def pair(t, cy):
            kA = 2 * t * SB
            kB = kA + SB
            fire(kB, sB, dB, gsemB)
            drain(kA, sA, dA, gsemA)
            process(kA, sA, dA)

            @pl.when(t < NSUP // 2 - 1)
            def _():
                fire(kA + 2 * SB, sA, dA, gsemA)

            drain(kB, sB, dB, gsemB)
            process(kB, sB, dB)
            return cy

        lax.fori_loop(0, NSUP // 2, pair, 0)
        plsc.subcore_barrier()
        pltpu.sync_copy(acc.at[pl.ds(s * RPT, RPT)], out_hbm.at[c, pl.ds(s * RPT, RPT)])

    return body


_edge1_body = _make_edge_body(_group1)
_edge2_body = _make_edge_body(_group2)


_SC_MESH = plsc.VectorSubcoreMesh(core_axis_name="c", subcore_axis_name="s")
_SC_PARAMS = pltpu.CompilerParams(
    needs_layout_passes=False, use_tc_tiling_on_sc=False)

_edge1 = functools.partial(
    pl.kernel,
    out_type=jax.ShapeDtypeStruct((NC, NPAD, ROW1), jnp.float32),
    mesh=_SC_MESH,
    compiler_params=_SC_PARAMS,
    scratch_types=[
        pltpu.VMEM((K1, CH), jnp.int32),
        pltpu.VMEM((K1, CH), jnp.int32),
        pltpu.VMEM((SB * CH, ROW1), jnp.float32),
        pltpu.VMEM((SB * CH, RDST), jnp.float32),
        pltpu.VMEM((SB * CH, ROW1), jnp.float32),
        pltpu.VMEM((SB * CH, RDST), jnp.float32),
        pltpu.VMEM_SHARED((NPAD, ROW1), jnp.float32),
        pltpu.SemaphoreType.DMA,
        pltpu.SemaphoreType.DMA,
    ],
)(_edge1_body)

_edge2 = functools.partial(
    pl.kernel,
    out_type=jax.ShapeDtypeStruct((NC, NPAD, ROW2), jnp.float32),
    mesh=_SC_MESH,
    compiler_params=_SC_PARAMS,
    scratch_types=[
        pltpu.VMEM((K1, CH), jnp.int32),
        pltpu.VMEM((K1, CH), jnp.int32),
        pltpu.VMEM((SB * CH, ROW2), jnp.float32),
        pltpu.VMEM((SB * CH, ROW2), jnp.float32),
        pltpu.VMEM((SB * CH, ROW2), jnp.float32),
        pltpu.VMEM((SB * CH, ROW2), jnp.float32),
        pltpu.VMEM_SHARED((NPAD, ROW2), jnp.float32),
        pltpu.SemaphoreType.DMA,
        pltpu.SemaphoreType.DMA,
    ],
)(_edge2_body)


def kernel(x, edge_index, W1, att_src1, att_dst1, bias1, W2, att_src2, att_dst2, bias2):
    f32 = jnp.float32
    eye8 = jnp.eye(H1, dtype=f32)
    As1 = (att_src1.reshape(H1, D1)[:, :, None] * eye8[:, None, :]).reshape(C1, H1)
    Ad1 = (att_dst1.reshape(H1, D1)[:, :, None] * eye8[:, None, :]).reshape(C1, H1)
    R = jnp.repeat(eye8, D1, axis=1)
    Ws2 = W2 @ att_src2.reshape(2, 1)
    Wd2 = W2 @ att_dst2.reshape(2, 1)
    xp = jnp.pad(x, ((0, NPAD - N), (0, 0)))

    loop = jnp.arange(N, dtype=jnp.int32)
    padv = N + jnp.arange(EPAD - EP_RAW, dtype=jnp.int32) % (NPAD - N)
    src = jnp.concatenate([edge_index[0], loop, padv]).reshape(TILES, K1, CH)
    dst = jnp.concatenate([edge_index[1], loop, padv]).reshape(TILES, K1, CH)

    zeros80 = jnp.zeros((NPAD, ROW1), f32)
    zeros16 = jnp.zeros((NPAD, ROW2), f32)

    grid = NPAD // BN
    t1, td = pl.pallas_call(
        _prep_kernel,
        grid=(grid,),
        in_specs=[
            pl.BlockSpec((BN, F_IN), lambda i: (i, 0)),
            pl.BlockSpec((F_IN, C1), lambda i: (0, 0)),
            pl.BlockSpec((C1, H1), lambda i: (0, 0)),
            pl.BlockSpec((C1, H1), lambda i: (0, 0)),
        ],
        out_specs=[
            pl.BlockSpec((BN, ROW1), lambda i: (i, 0)),
            pl.BlockSpec((BN, RDST), lambda i: (i, 0)),
        ],
        out_shape=[
            jax.ShapeDtypeStruct((NPAD, ROW1), f32),
            jax.ShapeDtypeStruct((NPAD, RDST), f32),
        ],
    )(xp, W1, As1, Ad1)

    parts1 = _edge1(t1, td, src, dst, zeros80)

    t2 = pl.pallas_call(
        _mid_kernel,
        grid=(grid,),
        in_specs=[
            pl.BlockSpec((BN, ROW1), lambda i: (i, 0)),
            pl.BlockSpec((BN, ROW1), lambda i: (i, 0)),
            pl.BlockSpec((1, C1), lambda i: (0, 0)),
            pl.BlockSpec((C1, 2), lambda i: (0, 0)),
            pl.BlockSpec((C1, 1), lambda i: (0, 0)),
            pl.BlockSpec((C1, 1), lambda i: (0, 0)),
            pl.BlockSpec((H1, C1), lambda i: (0, 0)),
        ],
        out_specs=pl.BlockSpec((BN, ROW2), lambda i: (i, 0)),
        out_shape=jax.ShapeDtypeStruct((NPAD, ROW2), f32),
    )(parts1[0], parts1[1], bias1.reshape(1, C1), W2, Ws2, Wd2, R)

    parts2 = _edge2(t2, t2, src, dst, zeros16)

    out = pl.pallas_call(
        _final_kernel,
        grid=(grid,),
        in_specs=[
            pl.BlockSpec((BN, ROW2), lambda i: (i, 0)),
            pl.BlockSpec((BN, ROW2), lambda i: (i, 0)),
            pl.BlockSpec((1, 2), lambda i: (0, 0)),
        ],
        out_specs=pl.BlockSpec((BN, 2), lambda i: (i, 0)),
        out_shape=jax.ShapeDtypeStruct((NPAD, 2), f32),
    )(parts2[0], parts2[1], bias2.reshape(1, 2))

    return out[:N]


# speedup vs baseline: 1.8702x; 1.0330x over previous
import functools

import jax
import jax.numpy as jnp
import numpy as np
from jax import lax
from jax.experimental import pallas as pl
from jax.experimental.pallas import tpu as pltpu
from jax.experimental.pallas import tpu_sc as plsc

N = 10000
F_IN = 128
H1 = 8
D1 = 8
C1 = H1 * D1
NPAD = 10240
BN = 1280
ROW1 = 80
ROW2 = 16
RDST = 16

NC = 2
NS = 16
TILES = NC * NS
CH = 128
EP_RAW = 320000 + N
SB = 2
K1 = 84
NSUP = K1 // SB
EPAD = TILES * CH * K1
RPT = NPAD // NS


def _prep_kernel(x_ref, w1_ref, as1_ref, ad1_ref, t1_ref, td_ref):
    h = jnp.dot(x_ref[...], w1_ref[...], preferred_element_type=jnp.float32)
    s = jnp.dot(h, as1_ref[...], preferred_element_type=jnp.float32)
    d = jnp.dot(h, ad1_ref[...], preferred_element_type=jnp.float32)
    z8 = jnp.zeros((h.shape[0], 8), jnp.float32)
    t1_ref[...] = jnp.concatenate([h, s, z8], axis=1)
    td_ref[...] = jnp.concatenate([d, z8], axis=1)


def _mid_kernel(p_ref, b1_ref, w2_ref, ws2_ref, wd2_ref, r_ref, t2_ref):
    a = p_ref[0] + p_ref[1]
    num = a[:, 0:C1]
    den = a[:, C1:C1 + H1]
    denr = jnp.dot(den, r_ref[...], preferred_element_type=jnp.float32)
    out1 = num / (denr + 1e-16) + b1_ref[...]
    g = jnp.where(out1 > 0, out1, jnp.exp(jnp.minimum(out1, 0.0)) - 1.0)
    h2 = jnp.dot(g, w2_ref[...], preferred_element_type=jnp.float32)
    s2 = jnp.dot(g, ws2_ref[...], preferred_element_type=jnp.float32)
    d2 = jnp.dot(g, wd2_ref[...], preferred_element_type=jnp.float32)
    z12 = jnp.zeros((a.shape[0], 12), jnp.float32)
    t2_ref[...] = jnp.concatenate([h2, s2, d2, z12], axis=1)


def _final_kernel(q_ref, b2_ref, o_ref):
    a = q_ref[0] + q_ref[1]
    num = a[:, 0:2]
    den = a[:, 2:3]
    o = num / (den + 1e-16) + b2_ref[...]
    m = jnp.max(o, axis=1, keepdims=True)
    lse = m + jnp.log(jnp.sum(jnp.exp(o - m), axis=1, keepdims=True))
    o_ref[...] = o - lse


def _leaky_exp(x):
    return jnp.exp(jnp.where(x >= 0, x, x * 0.2))


_U = 4


def _permute(vec, idx):
    return vec.at[idx].get(mode="promise_in_bounds")


def _group1(srows, drows, base):
    lanes = lax.iota(jnp.int32, 16)
    for u in range(_U):
        e = base + u
        al = srows[e, pl.ds(C1, 16)] + drows[e]
        p16 = jnp.exp(jnp.where(al >= 0.0, al, al * 0.2))
        srows[e, pl.ds(C1, 16)] = p16
        for v in range(4):
            prep = _permute(p16, lanes // 8 + 2 * v)
            srows[e, pl.ds(16 * v, 16)] = srows[e, pl.ds(16 * v, 16)] * prep


def _group2(srows, drows, base):
    lanes = lax.iota(jnp.int32, 16)
    for u in range(_U):
        e = base + u
        sr = srows[e]
        dr = drows[e]
        al = _permute(sr, lanes * 0 + 2) + _permute(dr, lanes * 0 + 3)
        p = jnp.exp(jnp.where(al >= 0.0, al, al * 0.2))
        srows[e] = jnp.where(lanes == 2, p, sr * p)


def _make_edge_body(group_fn):
    def body(tsrc_hbm, tdst_hbm, src_hbm, dst_hbm, zero_hbm, out_hbm,
             sidx, didx, sA, dA, sB, dB, acc, gsemA, gsemB):
        c = lax.axis_index("c")
        s = lax.axis_index("s")
        wid = s * NC + c
        pltpu.sync_copy(zero_hbm.at[pl.ds(s * RPT, RPT)], acc.at[pl.ds(s * RPT, RPT)])
        plsc.subcore_barrier()
        pltpu.sync_copy(src_hbm.at[wid], sidx)
        pltpu.sync_copy(dst_hbm.at[wid], didx)

        def fire(kk, srows, drows, gsem):
            for j in range(SB):
                pltpu.async_copy(tsrc_hbm.at[sidx.at[kk + j]],
                                 srows.at[pl.ds(j * CH, CH)], gsem)
                pltpu.async_copy(tdst_hbm.at[didx.at[kk + j]],
                                 drows.at[pl.ds(j * CH, CH)], gsem)

        def drain(kk, srows, drows, gsem):
            for j in range(SB):
                pltpu.make_async_copy(tsrc_hbm.at[sidx.at[kk + j]],
                                      srows.at[pl.ds(j * CH, CH)], gsem).wait()
                pltpu.make_async_copy(tdst_hbm.at[didx.at[kk + j]],
                                      drows.at[pl.ds(j * CH, CH)], gsem).wait()

        def process(kk, srows, drows):
            if group_fn is not None:
                lax.fori_loop(
                    0, SB * CH // _U,
                    lambda i, cy: (group_fn(srows, drows, i * _U), cy)[1], 0)
            for j in range(SB):
                pltpu.sync_copy(srows.at[pl.ds(j * CH, CH)],
                                acc.at[didx.at[kk + j]], add=True)

        fire(0, sA, dA, gsemA)

        def pair(t, cy):
            kA = 2 * t * SB
            kB = kA + SB
            fire(kB, sB, dB, gsemB)
            drain(kA, sA, dA, gsemA)
            process(kA, sA, dA)

            @pl.when(t < NSUP // 2 - 1)
            def _():
                fire(kA + 2 * SB, sA, dA, gsemA)

            drain(kB, sB, dB, gsemB)
            process(kB, sB, dB)
            return cy

        lax.fori_loop(0, NSUP // 2, pair, 0)
        plsc.subcore_barrier()
        pltpu.sync_copy(acc.at[pl.ds(s * RPT, RPT)], out_hbm.at[c, pl.ds(s * RPT, RPT)])

    return body


_edge1_body = _make_edge_body(_group1)
_edge2_body = _make_edge_body(_group2)


_SC_MESH = plsc.VectorSubcoreMesh(core_axis_name="c", subcore_axis_name="s")
_SC_PARAMS = pltpu.CompilerParams(
    needs_layout_passes=False, use_tc_tiling_on_sc=False)

_edge1 = functools.partial(
    pl.kernel,
    out_type=jax.ShapeDtypeStruct((NC, NPAD, ROW1), jnp.float32),
    mesh=_SC_MESH,
    compiler_params=_SC_PARAMS,
    scratch_types=[
        pltpu.VMEM((K1, CH), jnp.int32),
        pltpu.VMEM((K1, CH), jnp.int32),
        pltpu.VMEM((SB * CH, ROW1), jnp.float32),
        pltpu.VMEM((SB * CH, RDST), jnp.float32),
        pltpu.VMEM((SB * CH, ROW1), jnp.float32),
        pltpu.VMEM((SB * CH, RDST), jnp.float32),
        pltpu.VMEM_SHARED((NPAD, ROW1), jnp.float32),
        pltpu.SemaphoreType.DMA,
        pltpu.SemaphoreType.DMA,
    ],
)(_edge1_body)

_edge2 = functools.partial(
    pl.kernel,
    out_type=jax.ShapeDtypeStruct((NC, NPAD, ROW2), jnp.float32),
    mesh=_SC_MESH,
    compiler_params=_SC_PARAMS,
    scratch_types=[
        pltpu.VMEM((K1, CH), jnp.int32),
        pltpu.VMEM((K1, CH), jnp.int32),
        pltpu.VMEM((SB * CH, ROW2), jnp.float32),
        pltpu.VMEM((SB * CH, ROW2), jnp.float32),
        pltpu.VMEM((SB * CH, ROW2), jnp.float32),
        pltpu.VMEM((SB * CH, ROW2), jnp.float32),
        pltpu.VMEM_SHARED((NPAD, ROW2), jnp.float32),
        pltpu.SemaphoreType.DMA,
        pltpu.SemaphoreType.DMA,
    ],
)(_edge2_body)


def kernel(x, edge_index, W1, att_src1, att_dst1, bias1, W2, att_src2, att_dst2, bias2):
    f32 = jnp.float32
    eye8 = jnp.eye(H1, dtype=f32)
    As1 = (att_src1.reshape(H1, D1)[:, :, None] * eye8[:, None, :]).reshape(C1, H1)
    Ad1 = (att_dst1.reshape(H1, D1)[:, :, None] * eye8[:, None, :]).reshape(C1, H1)
    R = jnp.repeat(eye8, D1, axis=1)
    Ws2 = W2 @ att_src2.reshape(2, 1)
    Wd2 = W2 @ att_dst2.reshape(2, 1)
    xp = jnp.pad(x, ((0, NPAD - N), (0, 0)))

    loop = jnp.arange(N, dtype=jnp.int32)
    padv = N + jnp.arange(EPAD - EP_RAW, dtype=jnp.int32) % (NPAD - N)
    src = jnp.concatenate([edge_index[0], loop, padv]).reshape(TILES, K1, CH)
    dst = jnp.concatenate([edge_index[1], loop, padv]).reshape(TILES, K1, CH)

    zeros80 = jnp.zeros((NPAD, ROW1), f32)
    zeros16 = jnp.zeros((NPAD, ROW2), f32)

    grid = NPAD // BN
    t1, td = pl.pallas_call(
        _prep_kernel,
        grid=(grid,),
        in_specs=[
            pl.BlockSpec((BN, F_IN), lambda i: (i, 0)),
            pl.BlockSpec((F_IN, C1), lambda i: (0, 0)),
            pl.BlockSpec((C1, H1), lambda i: (0, 0)),
            pl.BlockSpec((C1, H1), lambda i: (0, 0)),
        ],
        out_specs=[
            pl.BlockSpec((BN, ROW1), lambda i: (i, 0)),
            pl.BlockSpec((BN, RDST), lambda i: (i, 0)),
        ],
        out_shape=[
            jax.ShapeDtypeStruct((NPAD, ROW1), f32),
            jax.ShapeDtypeStruct((NPAD, RDST), f32),
        ],
    )(xp, W1, As1, Ad1)

    parts1 = _edge1(t1, td, src, dst, zeros80)

    t2 = pl.pallas_call(
        _mid_kernel,
        grid=(grid,),
        in_specs=[
            pl.BlockSpec((NC, BN, ROW1), lambda i: (0, i, 0)),
            pl.BlockSpec((1, C1), lambda i: (0, 0)),
            pl.BlockSpec((C1, 2), lambda i: (0, 0)),
            pl.BlockSpec((C1, 1), lambda i: (0, 0)),
            pl.BlockSpec((C1, 1), lambda i: (0, 0)),
            pl.BlockSpec((H1, C1), lambda i: (0, 0)),
        ],
        out_specs=pl.BlockSpec((BN, ROW2), lambda i: (i, 0)),
        out_shape=jax.ShapeDtypeStruct((NPAD, ROW2), f32),
    )(parts1, bias1.reshape(1, C1), W2, Ws2, Wd2, R)

    parts2 = _edge2(t2, t2, src, dst, zeros16)

    out = pl.pallas_call(
        _final_kernel,
        grid=(grid,),
        in_specs=[
            pl.BlockSpec((NC, BN, ROW2), lambda i: (0, i, 0)),
            pl.BlockSpec((1, 2), lambda i: (0, 0)),
        ],
        out_specs=pl.BlockSpec((BN, 2), lambda i: (i, 0)),
        out_shape=jax.ShapeDtypeStruct((NPAD, 2), f32),
    )(parts2, bias2.reshape(1, 2))

    return out[:N]


# speedup vs baseline: 1.8708x; 1.0003x over previous
import functools

import jax
import jax.numpy as jnp
import numpy as np
from jax import lax
from jax.experimental import pallas as pl
from jax.experimental.pallas import tpu as pltpu
from jax.experimental.pallas import tpu_sc as plsc

N = 10000
F_IN = 128
H1 = 8
D1 = 8
C1 = H1 * D1
NPAD = 10240
BN = 1280
ROW1 = 80
ROW2 = 16
RDST = 16

NC = 2
NS = 16
TILES = NC * NS
CH = 128
EP_RAW = 320000 + N
SB = 2
K1 = 84
NSUP = K1 // SB
EPAD = TILES * CH * K1
RPT = NPAD // NS


def _prep_kernel(x_ref, w1_ref, as1_ref, ad1_ref, t1_ref, td_ref):
    h = jnp.dot(x_ref[...], w1_ref[...], preferred_element_type=jnp.float32)
    s = jnp.dot(h, as1_ref[...], preferred_element_type=jnp.float32)
    d = jnp.dot(h, ad1_ref[...], preferred_element_type=jnp.float32)
    z8 = jnp.zeros((h.shape[0], 8), jnp.float32)
    t1_ref[...] = jnp.concatenate([h, s, z8], axis=1)
    td_ref[...] = jnp.concatenate([d, z8], axis=1)


def _mid_kernel(p_ref, b1_ref, w2_ref, ws2_ref, wd2_ref, r_ref, t2_ref):
    a = p_ref[0] + p_ref[1]
    num = a[:, 0:C1]
    den = a[:, C1:C1 + H1]
    denr = jnp.dot(den, r_ref[...], preferred_element_type=jnp.float32)
    out1 = num / (denr + 1e-16) + b1_ref[...]
    g = jnp.where(out1 > 0, out1, jnp.exp(jnp.minimum(out1, 0.0)) - 1.0)
    h2 = jnp.dot(g, w2_ref[...], preferred_element_type=jnp.float32)
    s2 = jnp.dot(g, ws2_ref[...], preferred_element_type=jnp.float32)
    d2 = jnp.dot(g, wd2_ref[...], preferred_element_type=jnp.float32)
    z12 = jnp.zeros((a.shape[0], 12), jnp.float32)
    t2_ref[...] = jnp.concatenate([h2, s2, d2, z12], axis=1)


def _final_kernel(q_ref, b2_ref, o_ref):
    a = q_ref[0] + q_ref[1]
    num = a[:, 0:2]
    den = a[:, 2:3]
    o = num / (den + 1e-16) + b2_ref[...]
    m = jnp.max(o, axis=1, keepdims=True)
    lse = m + jnp.log(jnp.sum(jnp.exp(o - m), axis=1, keepdims=True))
    o_ref[...] = o - lse


def _leaky_exp(x):
    return jnp.exp(jnp.where(x >= 0, x, x * 0.2))


_U = 4


def _permute(vec, idx):
    return vec.at[idx].get(mode="promise_in_bounds")


def _group1(srows, drows, base):
    lanes = lax.iota(jnp.int32, 16)
    for u in range(_U):
        e = base + u
        al = srows[e, pl.ds(C1, 16)] + drows[e]
        p16 = jnp.exp(jnp.where(al >= 0.0, al, al * 0.2))
        srows[e, pl.ds(C1, 16)] = p16
        for v in range(4):
            prep = _permute(p16, lanes // 8 + 2 * v)
            srows[e, pl.ds(16 * v, 16)] = srows[e, pl.ds(16 * v, 16)] * prep


def _group2(srows, drows, base):
    lanes = lax.iota(jnp.int32, 16)
    for u in range(_U):
        e = base + u
        sr = srows[e]
        dr = drows[e]
        al = _permute(sr, lanes * 0 + 2) + _permute(dr, lanes * 0 + 3)
        p = jnp.exp(jnp.where(al >= 0.0, al, al * 0.2))
        srows[e] = jnp.where(lanes == 2, p, sr * p)


def _make_edge_body(group_fn):
    def body(tsrc_hbm, tdst_hbm, src_hbm, dst_hbm, zero_hbm, out_hbm,
             sidx, didx, sA, dA, sB, dB, acc, gsemA, gsemB):
        c = lax.axis_index("c")
        s = lax.axis_index("s")
        wid = s * NC + c
        pltpu.sync_copy(zero_hbm.at[pl.ds(s * RPT, RPT)], acc.at[pl.ds(s * RPT, RPT)])
        plsc.subcore_barrier()
        pltpu.sync_copy(src_hbm.at[wid], sidx)
        pltpu.sync_copy(dst_hbm.at[wid], didx)

        def fire(kk, srows, drows, gsem):
            for j in range(SB):
                pltpu.async_copy(tsrc_hbm.at[sidx.at[kk + j]],
                                 srows.at[pl.ds(j * CH, CH)], gsem)
                pltpu.async_copy(tdst_hbm.at[didx.at[kk + j]],
                                 drows.at[pl.ds(j * CH, CH)], gsem)

        def drain(kk, srows, drows, gsem):
            for j in range(SB):
                pltpu.make_async_copy(tsrc_hbm.at[sidx.at[kk + j]],
                                      srows.at[pl.ds(j * CH, CH)], gsem).wait()
                pltpu.make_async_copy(tdst_hbm.at[didx.at[kk + j]],
                                      drows.at[pl.ds(j * CH, CH)], gsem).wait()

        def process(kk, srows, drows):
            lax.fori_loop(
                0, SB * CH // _U,
                lambda i, cy: (group_fn(srows, drows, i * _U), cy)[1], 0)
            for j in range(SB):
                pltpu.sync_copy(srows.at[pl.ds(j * CH, CH)],
                                acc.at[didx.at[kk + j]], add=True)

        fire(0, sA, dA, gsemA)

        def pair(t, cy):
            kA = 2 * t * SB
            kB = kA + SB
            fire(kB, sB, dB, gsemB)
            drain(kA, sA, dA, gsemA)
            process(kA, sA, dA)

            @pl.when(t < NSUP // 2 - 1)
            def _():
                fire(kA + 2 * SB, sA, dA, gsemA)

            drain(kB, sB, dB, gsemB)
            process(kB, sB, dB)
            return cy

        lax.fori_loop(0, NSUP // 2, pair, 0)
        plsc.subcore_barrier()
        pltpu.sync_copy(acc.at[pl.ds(s * RPT, RPT)], out_hbm.at[c, pl.ds(s * RPT, RPT)])

    return body


_edge1_body = _make_edge_body(_group1)
_edge2_body = _make_edge_body(_group2)


_SC_MESH = plsc.VectorSubcoreMesh(core_axis_name="c", subcore_axis_name="s")
_SC_PARAMS = pltpu.CompilerParams(
    needs_layout_passes=False, use_tc_tiling_on_sc=False)

_edge1 = functools.partial(
    pl.kernel,
    out_type=jax.ShapeDtypeStruct((NC, NPAD, ROW1), jnp.float32),
    mesh=_SC_MESH,
    compiler_params=_SC_PARAMS,
    scratch_types=[
        pltpu.VMEM((K1, CH), jnp.int32),
        pltpu.VMEM((K1, CH), jnp.int32),
        pltpu.VMEM((SB * CH, ROW1), jnp.float32),
        pltpu.VMEM((SB * CH, RDST), jnp.float32),
        pltpu.VMEM((SB * CH, ROW1), jnp.float32),
        pltpu.VMEM((SB * CH, RDST), jnp.float32),
        pltpu.VMEM_SHARED((NPAD, ROW1), jnp.float32),
        pltpu.SemaphoreType.DMA,
        pltpu.SemaphoreType.DMA,
    ],
)(_edge1_body)

_edge2 = functools.partial(
    pl.kernel,
    out_type=jax.ShapeDtypeStruct((NC, NPAD, ROW2), jnp.float32),
    mesh=_SC_MESH,
    compiler_params=_SC_PARAMS,
    scratch_types=[
        pltpu.VMEM((K1, CH), jnp.int32),
        pltpu.VMEM((K1, CH), jnp.int32),
        pltpu.VMEM((SB * CH, ROW2), jnp.float32),
        pltpu.VMEM((SB * CH, ROW2), jnp.float32),
        pltpu.VMEM((SB * CH, ROW2), jnp.float32),
        pltpu.VMEM((SB * CH, ROW2), jnp.float32),
        pltpu.VMEM_SHARED((NPAD, ROW2), jnp.float32),
        pltpu.SemaphoreType.DMA,
        pltpu.SemaphoreType.DMA,
    ],
)(_edge2_body)


def kernel(x, edge_index, W1, att_src1, att_dst1, bias1, W2, att_src2, att_dst2, bias2):
    f32 = jnp.float32
    eye8 = jnp.eye(H1, dtype=f32)
    As1 = (att_src1.reshape(H1, D1)[:, :, None] * eye8[:, None, :]).reshape(C1, H1)
    Ad1 = (att_dst1.reshape(H1, D1)[:, :, None] * eye8[:, None, :]).reshape(C1, H1)
    R = jnp.repeat(eye8, D1, axis=1)
    Ws2 = W2 @ att_src2.reshape(2, 1)
    Wd2 = W2 @ att_dst2.reshape(2, 1)
    xp = jnp.pad(x, ((0, NPAD - N), (0, 0)))

    loop = jnp.arange(N, dtype=jnp.int32)
    padv = N + jnp.arange(EPAD - EP_RAW, dtype=jnp.int32) % (NPAD - N)
    src = jnp.concatenate([edge_index[0], loop, padv]).reshape(TILES, K1, CH)
    dst = jnp.concatenate([edge_index[1], loop, padv]).reshape(TILES, K1, CH)

    zeros80 = jnp.zeros((NPAD, ROW1), f32)
    zeros16 = jnp.zeros((NPAD, ROW2), f32)

    grid = NPAD // BN
    t1, td = pl.pallas_call(
        _prep_kernel,
        grid=(grid,),
        in_specs=[
            pl.BlockSpec((BN, F_IN), lambda i: (i, 0)),
            pl.BlockSpec((F_IN, C1), lambda i: (0, 0)),
            pl.BlockSpec((C1, H1), lambda i: (0, 0)),
            pl.BlockSpec((C1, H1), lambda i: (0, 0)),
        ],
        out_specs=[
            pl.BlockSpec((BN, ROW1), lambda i: (i, 0)),
            pl.BlockSpec((BN, RDST), lambda i: (i, 0)),
        ],
        out_shape=[
            jax.ShapeDtypeStruct((NPAD, ROW1), f32),
            jax.ShapeDtypeStruct((NPAD, RDST), f32),
        ],
    )(xp, W1, As1, Ad1)

    parts1 = _edge1(t1, td, src, dst, zeros80)

    t2 = pl.pallas_call(
        _mid_kernel,
        grid=(grid,),
        in_specs=[
            pl.BlockSpec((NC, BN, ROW1), lambda i: (0, i, 0)),
            pl.BlockSpec((1, C1), lambda i: (0, 0)),
            pl.BlockSpec((C1, 2), lambda i: (0, 0)),
            pl.BlockSpec((C1, 1), lambda i: (0, 0)),
            pl.BlockSpec((C1, 1), lambda i: (0, 0)),
            pl.BlockSpec((H1, C1), lambda i: (0, 0)),
        ],
        out_specs=pl.BlockSpec((BN, ROW2), lambda i: (i, 0)),
        out_shape=jax.ShapeDtypeStruct((NPAD, ROW2), f32),
    )(parts1, bias1.reshape(1, C1), W2, Ws2, Wd2, R)

    parts2 = _edge2(t2, t2, src, dst, zeros16)

    out = pl.pallas_call(
        _final_kernel,
        grid=(grid,),
        in_specs=[
            pl.BlockSpec((NC, BN, ROW2), lambda i: (0, i, 0)),
            pl.BlockSpec((1, 2), lambda i: (0, 0)),
        ],
        out_specs=pl.BlockSpec((BN, 2), lambda i: (i, 0)),
        out_shape=jax.ShapeDtypeStruct((NPAD, 2), f32),
    )(parts2, bias2.reshape(1, 2))

    return out[:N]
